# Initial kernel scaffold; baseline (speedup 1.0000x reference)
import functools

import jax
import jax.numpy as jnp
from jax import lax
from jax.experimental import pallas as pl
from jax.experimental.pallas import tpu as pltpu
from jax.experimental.pallas import tpu_sc as plsc

N = 10000
M = 5000
E = 320000
D = 128

NC, NS = 2, 16
NW = NC * NS
CHUNK = 128
CPW = 80
E_PAD = NW * CPW * CHUNK

M_PAD = 5120


def _make_seg_sum(table_rows: int, acc_rows: int, zrows: int):
    stripe = acc_rows // NS
    nz = stripe // zrows
    mesh = plsc.VectorSubcoreMesh(core_axis_name="c", subcore_axis_name="s")

    @functools.partial(
        pl.kernel,
        out_type=jax.ShapeDtypeStruct((NC, acc_rows, D), jnp.float32),
        mesh=mesh,
        scratch_types=[
            pltpu.VMEM((CPW, CHUNK), jnp.int32),
            pltpu.VMEM((CPW, CHUNK), jnp.int32),
            pltpu.VMEM((CHUNK, D), jnp.float32),
            pltpu.VMEM((CHUNK, D), jnp.float32),
            pltpu.VMEM((zrows, D), jnp.float32),
            pltpu.VMEM_SHARED((acc_rows, D), jnp.float32),
            pltpu.SemaphoreType.DMA,
            pltpu.SemaphoreType.DMA,
        ],
    )
    def seg(table, gidx, sidx, out, gidx_v, sidx_v, buf0, buf1, zbuf, acc, sem0, sem1):
        c = lax.axis_index("c")
        s = lax.axis_index("s")
        wid = c * NS + s
        zero16 = jnp.zeros((16,), jnp.float32)

        def zfill(i, carry):
            for j in range(D // 16):
                zbuf[i, pl.ds(j * 16, 16)] = zero16
            return carry

        lax.fori_loop(0, zrows, zfill, 0)

        row0 = s * stripe

        def zcopy(i, carry):
            pltpu.sync_copy(zbuf, acc.at[pl.ds(row0 + i * zrows, zrows)])
            return carry

        lax.fori_loop(0, nz, zcopy, 0)
        plsc.subcore_barrier()

        pltpu.sync_copy(gidx.at[wid], gidx_v)
        pltpu.sync_copy(sidx.at[wid], sidx_v)

        pltpu.async_copy(table.at[gidx_v.at[0]], buf0, sem0)

        def body(jj, carry):
            j0 = 2 * jj
            j1 = j0 + 1
            pltpu.async_copy(table.at[gidx_v.at[j1]], buf1, sem1)
            pltpu.make_async_copy(table.at[gidx_v.at[j0]], buf0, sem0).wait()
            pltpu.sync_copy(buf0, acc.at[sidx_v.at[j0]], add=True)

            @pl.when(j1 + 1 < CPW)
            def _():
                pltpu.async_copy(table.at[gidx_v.at[j1 + 1]], buf0, sem0)

            pltpu.make_async_copy(table.at[gidx_v.at[j1]], buf1, sem1).wait()
            pltpu.sync_copy(buf1, acc.at[sidx_v.at[j1]], add=True)
            return carry

        lax.fori_loop(0, CPW // 2, body, 0)
        plsc.subcore_barrier()
        pltpu.sync_copy(acc.at[pl.ds(row0, stripe)], out.at[c, pl.ds(row0, stripe)])

    return seg


_seg_edges = _make_seg_sum(N, M_PAD, 32)
_seg_nodes = _make_seg_sum(M_PAD, N, 25)


BN_BLK = 400


def _bn_body(x_ref, g_ref, b_ref, dv_ref, o_ref, acc_ref):
    p = pl.program_id(0)
    i = pl.program_id(1)

    @pl.when((p == 0) & (i == 0))
    def _():
        acc_ref[...] = jnp.zeros_like(acc_ref)

    @pl.when(p == 0)
    def _():
        x = x_ref[...]
        acc_ref[0:1, :] += jnp.sum(x, axis=0, keepdims=True)
        acc_ref[1:2, :] += jnp.sum(x * x, axis=0, keepdims=True)

    @pl.when(p == 1)
    def _():
        x = x_ref[...]
        mean = acc_ref[0:1, :] * (1.0 / N)
        var = acc_ref[1:2, :] * (1.0 / N) - mean * mean
        inv = lax.rsqrt(var + 1e-5)
        o_ref[...] = ((x - mean) * inv * g_ref[...] + b_ref[...]) * dv_ref[...]


def _bn(X, gamma, beta, dv):
    return pl.pallas_call(
        _bn_body,
        grid=(2, N // BN_BLK),
        in_specs=[
            pl.BlockSpec((BN_BLK, D), lambda p, i: (i, 0)),
            pl.BlockSpec((1, D), lambda p, i: (0, 0)),
            pl.BlockSpec((1, D), lambda p, i: (0, 0)),
            pl.BlockSpec((BN_BLK, 1), lambda p, i: (i, 0)),
        ],
        out_specs=pl.BlockSpec((BN_BLK, D), lambda p, i: (i, 0)),
        out_shape=jax.ShapeDtypeStruct((N, D), jnp.float32),
        scratch_shapes=[pltpu.VMEM((2, D), jnp.float32)],
    )(X, gamma.reshape(1, D), beta.reshape(1, D), dv.reshape(N, 1))


CB_BLK = 512


def _combine_body(y_ref, ds_ref, de_ref, o_ref):
    y = y_ref[0] + y_ref[1]
    o_ref[...] = jnp.maximum(y * ds_ref[...], 0.0) * de_ref[...]


def _combine(Yp, desum_p, de_p):
    return pl.pallas_call(
        _combine_body,
        grid=(M_PAD // CB_BLK,),
        in_specs=[
            pl.BlockSpec((2, CB_BLK, D), lambda i: (0, i, 0)),
            pl.BlockSpec((CB_BLK, 1), lambda i: (i, 0)),
            pl.BlockSpec((CB_BLK, 1), lambda i: (i, 0)),
        ],
        out_specs=pl.BlockSpec((CB_BLK, D), lambda i: (i, 0)),
        out_shape=jax.ShapeDtypeStruct((M_PAD, D), jnp.float32),
    )(Yp, desum_p, de_p)


F_BLK = 400


def _final_body(x_ref, dv_ref, w_ref, b_ref, o_ref):
    x = (x_ref[0] + x_ref[1]) * dv_ref[...]
    o_ref[...] = jnp.dot(x, w_ref[...], preferred_element_type=jnp.float32) + b_ref[...]


def _final(X2p, dvs, WvT, bv):
    return pl.pallas_call(
        _final_body,
        grid=(N // F_BLK,),
        in_specs=[
            pl.BlockSpec((2, F_BLK, D), lambda i: (0, i, 0)),
            pl.BlockSpec((F_BLK, 1), lambda i: (i, 0)),
            pl.BlockSpec((D, D), lambda i: (0, 0)),
            pl.BlockSpec((1, D), lambda i: (0, 0)),
        ],
        out_specs=pl.BlockSpec((F_BLK, D), lambda i: (i, 0)),
        out_shape=jax.ShapeDtypeStruct((N, D), jnp.float32),
    )(X2p, dvs, WvT, bv)


@jax.jit
def kernel(X_origin, node_idx, edge_idx, gamma, beta, dv, dv_sum, de, de_sum, Wv, bv):
    node32 = node_idx.astype(jnp.int32)
    edge32 = edge_idx.astype(jnp.int32)
    pad = E_PAD - E
    gA = jnp.concatenate([node32, jnp.zeros((pad,), jnp.int32)]).reshape(NW, CPW, CHUNK)
    sA = jnp.concatenate([edge32, jnp.full((pad,), M, jnp.int32)]).reshape(NW, CPW, CHUNK)
    gB = jnp.concatenate([edge32, jnp.full((pad,), M, jnp.int32)]).reshape(NW, CPW, CHUNK)
    sB = jnp.concatenate([node32, jnp.zeros((pad,), jnp.int32)]).reshape(NW, CPW, CHUNK)

    desum_p = jnp.pad(de_sum, (0, M_PAD - M)).reshape(M_PAD, 1)
    de_p = jnp.pad(de, (0, M_PAD - M)).reshape(M_PAD, 1)

    Xs = _bn(X_origin, gamma, beta, dv)
    Yp = _seg_edges(Xs, gA, sA)
    Yf = _combine(Yp, desum_p, de_p)
    X2p = _seg_nodes(Yf, gB, sB)
    return _final(X2p, dv_sum.reshape(N, 1), Wv.T, bv.reshape(1, D))


# speedup vs baseline: 2.7094x; 2.7094x over previous
import functools

import jax
import jax.numpy as jnp
from jax import lax
from jax.experimental import pallas as pl
from jax.experimental.pallas import tpu as pltpu
from jax.experimental.pallas import tpu_sc as plsc

N = 10000
M = 5000
E = 320000
D = 128

NC, NS = 2, 16
NW = NC * NS
E_PAD = 327680
EPW = E_PAD // NW

M_PAD = 5120
N_PAD = 10240


def _make_seg_sum(table_rows: int, acc_rows: int, nbuf: int, zrows: int = 32):
    stripe = acc_rows // NS
    nz = stripe // zrows
    chunk = 128
    cpw = EPW // chunk
    mesh = plsc.VectorSubcoreMesh(core_axis_name="c", subcore_axis_name="s")

    @functools.partial(
        pl.kernel,
        out_type=jax.ShapeDtypeStruct((NC, acc_rows, D), jnp.float32),
        mesh=mesh,
        scratch_types=[
            pltpu.VMEM((cpw, chunk), jnp.int32),
            pltpu.VMEM((cpw, chunk), jnp.int32),
            [pltpu.VMEM((chunk, D), jnp.float32) for _ in range(nbuf)],
            pltpu.VMEM((zrows, D), jnp.float32),
            pltpu.VMEM_SHARED((acc_rows, D), jnp.float32),
            pltpu.SemaphoreType.DMA,
            pltpu.SemaphoreType.DMA,
        ],
    )
    def seg(table, gidx, sidx, out, gidx_v, sidx_v, bufs, zbuf, acc, sem0, sem1):
        c = lax.axis_index("c")
        s = lax.axis_index("s")
        wid = c * NS + s
        zero16 = jnp.zeros((16,), jnp.float32)

        def zfill(i, carry):
            for j in range(D // 16):
                zbuf[i, pl.ds(j * 16, 16)] = zero16
            return carry

        lax.fori_loop(0, zrows, zfill, 0)

        row0 = s * stripe

        def zcopy(i, carry):
            pltpu.sync_copy(zbuf, acc.at[pl.ds(row0 + i * zrows, zrows)])
            return carry

        lax.fori_loop(0, nz, zcopy, 0)
        plsc.subcore_barrier()

        pltpu.sync_copy(gidx.at[wid], gidx_v)
        pltpu.sync_copy(sidx.at[wid], sidx_v)

        if nbuf == 2:
            buf0, buf1 = bufs
            pltpu.async_copy(table.at[gidx_v.at[0]], buf0, sem0)

            def body(jj, carry):
                j0 = 2 * jj
                j1 = j0 + 1
                pltpu.async_copy(table.at[gidx_v.at[j1]], buf1, sem1)
                pltpu.make_async_copy(table.at[gidx_v.at[j0]], buf0, sem0).wait()
                pltpu.sync_copy(buf0, acc.at[sidx_v.at[j0]], add=True)

                @pl.when(j1 + 1 < cpw)
                def _():
                    pltpu.async_copy(table.at[gidx_v.at[j1 + 1]], buf0, sem0)

                pltpu.make_async_copy(table.at[gidx_v.at[j1]], buf1, sem1).wait()
                pltpu.sync_copy(buf1, acc.at[sidx_v.at[j1]], add=True)
                return carry

            lax.fori_loop(0, cpw // 2, body, 0)
        else:
            (buf0,) = bufs

            def body(j, carry):
                pltpu.async_copy(table.at[gidx_v.at[j]], buf0, sem0)
                pltpu.make_async_copy(table.at[gidx_v.at[j]], buf0, sem0).wait()
                pltpu.sync_copy(buf0, acc.at[sidx_v.at[j]], add=True)
                return carry

            lax.fori_loop(0, cpw, body, 0)
        plsc.subcore_barrier()
        pltpu.sync_copy(acc.at[pl.ds(row0, stripe)], out.at[c, pl.ds(row0, stripe)])

    return seg


_seg_edges = _make_seg_sum(N, M_PAD, 2)
_seg_nodes = _make_seg_sum(M_PAD, N_PAD, 1)


BN_BLK = 400


def _bn_body(x_ref, g_ref, b_ref, dv_ref, o_ref, acc_ref):
    p = pl.program_id(0)
    i = pl.program_id(1)

    @pl.when((p == 0) & (i == 0))
    def _():
        acc_ref[...] = jnp.zeros_like(acc_ref)

    @pl.when(p == 0)
    def _():
        x = x_ref[...]
        acc_ref[0:1, :] += jnp.sum(x, axis=0, keepdims=True)
        acc_ref[1:2, :] += jnp.sum(x * x, axis=0, keepdims=True)

    @pl.when(p == 1)
    def _():
        x = x_ref[...]
        mean = acc_ref[0:1, :] * (1.0 / N)
        var = acc_ref[1:2, :] * (1.0 / N) - mean * mean
        inv = lax.rsqrt(var + 1e-5)
        o_ref[...] = ((x - mean) * inv * g_ref[...] + b_ref[...]) * dv_ref[...]


def _bn(X, gamma, beta, dv):
    return pl.pallas_call(
        _bn_body,
        grid=(2, N // BN_BLK),
        in_specs=[
            pl.BlockSpec((BN_BLK, D), lambda p, i: (i, 0)),
            pl.BlockSpec((1, D), lambda p, i: (0, 0)),
            pl.BlockSpec((1, D), lambda p, i: (0, 0)),
            pl.BlockSpec((BN_BLK, 1), lambda p, i: (i, 0)),
        ],
        out_specs=pl.BlockSpec((BN_BLK, D), lambda p, i: (i, 0)),
        out_shape=jax.ShapeDtypeStruct((N, D), jnp.float32),
        scratch_shapes=[pltpu.VMEM((2, D), jnp.float32)],
    )(X, gamma.reshape(1, D), beta.reshape(1, D), dv.reshape(N, 1))


CB_BLK = 512


def _combine_body(y_ref, ds_ref, de_ref, o_ref):
    y = y_ref[0] + y_ref[1]
    o_ref[...] = jnp.maximum(y * ds_ref[...], 0.0) * de_ref[...]


def _combine(Yp, desum_p, de_p):
    return pl.pallas_call(
        _combine_body,
        grid=(M_PAD // CB_BLK,),
        in_specs=[
            pl.BlockSpec((2, CB_BLK, D), lambda i: (0, i, 0)),
            pl.BlockSpec((CB_BLK, 1), lambda i: (i, 0)),
            pl.BlockSpec((CB_BLK, 1), lambda i: (i, 0)),
        ],
        out_specs=pl.BlockSpec((CB_BLK, D), lambda i: (i, 0)),
        out_shape=jax.ShapeDtypeStruct((M_PAD, D), jnp.float32),
    )(Yp, desum_p, de_p)


F_BLK = 400


def _final_body(x_ref, dv_ref, w_ref, b_ref, o_ref):
    x = (x_ref[0] + x_ref[1]) * dv_ref[...]
    o_ref[...] = jnp.dot(x, w_ref[...], preferred_element_type=jnp.float32) + b_ref[...]


def _final(X2p, dvs, WvT, bv):
    return pl.pallas_call(
        _final_body,
        grid=(N // F_BLK,),
        in_specs=[
            pl.BlockSpec((2, F_BLK, D), lambda i: (0, i, 0)),
            pl.BlockSpec((F_BLK, 1), lambda i: (i, 0)),
            pl.BlockSpec((D, D), lambda i: (0, 0)),
            pl.BlockSpec((1, D), lambda i: (0, 0)),
        ],
        out_specs=pl.BlockSpec((F_BLK, D), lambda i: (i, 0)),
        out_shape=jax.ShapeDtypeStruct((N, D), jnp.float32),
    )(X2p, dvs, WvT, bv)


@jax.jit
def kernel(X_origin, node_idx, edge_idx, gamma, beta, dv, dv_sum, de, de_sum, Wv, bv):
    node32 = node_idx.astype(jnp.int32)
    edge32 = edge_idx.astype(jnp.int32)
    pad = E_PAD - E
    gA = jnp.concatenate([node32, jnp.zeros((pad,), jnp.int32)]).reshape(NW, EPW // 128, 128)
    sA = jnp.concatenate([edge32, jnp.full((pad,), M, jnp.int32)]).reshape(NW, EPW // 128, 128)
    gB = jnp.concatenate([edge32, jnp.full((pad,), M, jnp.int32)]).reshape(NW, EPW // 128, 128)
    sB = jnp.concatenate([node32, jnp.zeros((pad,), jnp.int32)]).reshape(NW, EPW // 128, 128)

    desum_p = jnp.pad(de_sum, (0, M_PAD - M)).reshape(M_PAD, 1)
    de_p = jnp.pad(de, (0, M_PAD - M)).reshape(M_PAD, 1)

    Xs = _bn(X_origin, gamma, beta, dv)
    Yp = _seg_edges(Xs, gA, sA)
    Yf = _combine(Yp, desum_p, de_p)
    X2p = _seg_nodes(Yf, gB, sB)
    return _final(X2p, dv_sum.reshape(N, 1), Wv.T, bv.reshape(1, D))


# speedup vs baseline: 2.8511x; 1.0523x over previous
import functools

import jax
import jax.numpy as jnp
from jax import lax
from jax.experimental import pallas as pl
from jax.experimental.pallas import tpu as pltpu
from jax.experimental.pallas import tpu_sc as plsc

N = 10000
M = 5000
E = 320000
D = 128

NC, NS = 2, 16
NW = NC * NS
E_PAD = 327680
EPW = E_PAD // NW

M_PAD = 5120
N_PAD = 10240


def _make_seg_sum(table_rows: int, acc_rows: int, nbuf: int, halves: int, zrows: int = 8):
    stripe = acc_rows // NS
    nz = stripe // zrows
    chunk = 128
    cpw = EPW // chunk
    cph = cpw // halves
    assert cph % nbuf == 0
    mesh = plsc.VectorSubcoreMesh(core_axis_name="c", subcore_axis_name="s")

    @functools.partial(
        pl.kernel,
        out_type=jax.ShapeDtypeStruct((NC, acc_rows, D), jnp.float32),
        mesh=mesh,
        scratch_types=[
            pltpu.VMEM((cph, chunk), jnp.int32),
            pltpu.VMEM((cph, chunk), jnp.int32),
            [pltpu.VMEM((chunk, D), jnp.float32) for _ in range(nbuf)],
            pltpu.VMEM((zrows, D), jnp.float32),
            pltpu.VMEM_SHARED((acc_rows, D), jnp.float32),
            [pltpu.SemaphoreType.DMA for _ in range(nbuf)],
        ],
    )
    def seg(table, gidx, sidx, out, gidx_v, sidx_v, bufs, zbuf, acc, sems):
        c = lax.axis_index("c")
        s = lax.axis_index("s")
        wid = c * NS + s
        zero16 = jnp.zeros((16,), jnp.float32)

        def zfill(i, carry):
            for j in range(D // 16):
                zbuf[i, pl.ds(j * 16, 16)] = zero16
            return carry

        lax.fori_loop(0, zrows, zfill, 0)

        row0 = s * stripe

        def zcopy(i, carry):
            pltpu.sync_copy(zbuf, acc.at[pl.ds(row0 + i * zrows, zrows)])
            return carry

        lax.fori_loop(0, nz, zcopy, 0)
        plsc.subcore_barrier()

        for h in range(halves):
            pltpu.sync_copy(gidx.at[wid, pl.ds(h * cph, cph)], gidx_v)
            pltpu.sync_copy(sidx.at[wid, pl.ds(h * cph, cph)], sidx_v)
            for k in range(nbuf - 1):
                pltpu.async_copy(table.at[gidx_v.at[k]], bufs[k], sems[k])

            def body(jj, carry):
                base = nbuf * jj
                pltpu.async_copy(table.at[gidx_v.at[base + nbuf - 1]],
                                 bufs[nbuf - 1], sems[nbuf - 1])
                for k in range(nbuf):
                    jk = base + k
                    pltpu.make_async_copy(table.at[gidx_v.at[jk]], bufs[k], sems[k]).wait()
                    pltpu.sync_copy(bufs[k], acc.at[sidx_v.at[jk]], add=True)
                    if k < nbuf - 1:
                        @pl.when(jk + nbuf < cph)
                        def _():
                            pltpu.async_copy(table.at[gidx_v.at[jk + nbuf]],
                                             bufs[k], sems[k])
                return carry

            lax.fori_loop(0, cph // nbuf, body, 0)
        plsc.subcore_barrier()
        pltpu.sync_copy(acc.at[pl.ds(row0, stripe)], out.at[c, pl.ds(row0, stripe)])

    return seg


_seg_edges = _make_seg_sum(N, M_PAD, 4, 1)
_seg_nodes = _make_seg_sum(M_PAD, N_PAD, 2, 2)


BN_BLK = 400


def _bn_body(x_ref, g_ref, b_ref, dv_ref, o_ref, acc_ref):
    p = pl.program_id(0)
    i = pl.program_id(1)

    @pl.when((p == 0) & (i == 0))
    def _():
        acc_ref[...] = jnp.zeros_like(acc_ref)

    @pl.when(p == 0)
    def _():
        x = x_ref[...]
        acc_ref[0:1, :] += jnp.sum(x, axis=0, keepdims=True)
        acc_ref[1:2, :] += jnp.sum(x * x, axis=0, keepdims=True)

    @pl.when(p == 1)
    def _():
        x = x_ref[...]
        mean = acc_ref[0:1, :] * (1.0 / N)
        var = acc_ref[1:2, :] * (1.0 / N) - mean * mean
        inv = lax.rsqrt(var + 1e-5)
        o_ref[...] = ((x - mean) * inv * g_ref[...] + b_ref[...]) * dv_ref[...]


def _bn(X, gamma, beta, dv):
    return pl.pallas_call(
        _bn_body,
        grid=(2, N // BN_BLK),
        in_specs=[
            pl.BlockSpec((BN_BLK, D), lambda p, i: (i, 0)),
            pl.BlockSpec((1, D), lambda p, i: (0, 0)),
            pl.BlockSpec((1, D), lambda p, i: (0, 0)),
            pl.BlockSpec((BN_BLK, 1), lambda p, i: (i, 0)),
        ],
        out_specs=pl.BlockSpec((BN_BLK, D), lambda p, i: (i, 0)),
        out_shape=jax.ShapeDtypeStruct((N, D), jnp.float32),
        scratch_shapes=[pltpu.VMEM((2, D), jnp.float32)],
    )(X, gamma.reshape(1, D), beta.reshape(1, D), dv.reshape(N, 1))


CB_BLK = 512


def _combine_body(y_ref, ds_ref, de_ref, o_ref):
    y = y_ref[0] + y_ref[1]
    o_ref[...] = jnp.maximum(y * ds_ref[...], 0.0) * de_ref[...]


def _combine(Yp, desum_p, de_p):
    return pl.pallas_call(
        _combine_body,
        grid=(M_PAD // CB_BLK,),
        in_specs=[
            pl.BlockSpec((2, CB_BLK, D), lambda i: (0, i, 0)),
            pl.BlockSpec((CB_BLK, 1), lambda i: (i, 0)),
            pl.BlockSpec((CB_BLK, 1), lambda i: (i, 0)),
        ],
        out_specs=pl.BlockSpec((CB_BLK, D), lambda i: (i, 0)),
        out_shape=jax.ShapeDtypeStruct((M_PAD, D), jnp.float32),
    )(Yp, desum_p, de_p)


F_BLK = 400


def _final_body(x_ref, dv_ref, w_ref, b_ref, o_ref):
    x = (x_ref[0] + x_ref[1]) * dv_ref[...]
    o_ref[...] = jnp.dot(x, w_ref[...], preferred_element_type=jnp.float32) + b_ref[...]


def _final(X2p, dvs, WvT, bv):
    return pl.pallas_call(
        _final_body,
        grid=(N // F_BLK,),
        in_specs=[
            pl.BlockSpec((2, F_BLK, D), lambda i: (0, i, 0)),
            pl.BlockSpec((F_BLK, 1), lambda i: (i, 0)),
            pl.BlockSpec((D, D), lambda i: (0, 0)),
            pl.BlockSpec((1, D), lambda i: (0, 0)),
        ],
        out_specs=pl.BlockSpec((F_BLK, D), lambda i: (i, 0)),
        out_shape=jax.ShapeDtypeStruct((N, D), jnp.float32),
    )(X2p, dvs, WvT, bv)


@jax.jit
def kernel(X_origin, node_idx, edge_idx, gamma, beta, dv, dv_sum, de, de_sum, Wv, bv):
    node32 = node_idx.astype(jnp.int32)
    edge32 = edge_idx.astype(jnp.int32)
    pad = E_PAD - E
    gA = jnp.concatenate([node32, jnp.zeros((pad,), jnp.int32)]).reshape(NW, EPW // 128, 128)
    sA = jnp.concatenate([edge32, jnp.full((pad,), M, jnp.int32)]).reshape(NW, EPW // 128, 128)
    gB = jnp.concatenate([edge32, jnp.full((pad,), M, jnp.int32)]).reshape(NW, EPW // 128, 128)
    sB = jnp.concatenate([node32, jnp.zeros((pad,), jnp.int32)]).reshape(NW, EPW // 128, 128)

    desum_p = jnp.pad(de_sum, (0, M_PAD - M)).reshape(M_PAD, 1)
    de_p = jnp.pad(de, (0, M_PAD - M)).reshape(M_PAD, 1)

    Xs = _bn(X_origin, gamma, beta, dv)
    Yp = _seg_edges(Xs, gA, sA)
    Yf = _combine(Yp, desum_p, de_p)
    X2p = _seg_nodes(Yf, gB, sB)
    return _final(X2p, dv_sum.reshape(N, 1), Wv.T, bv.reshape(1, D))
